# baseline (speedup 1.0000x reference)
import functools

import jax
import jax.numpy as jnp
from jax import lax
from jax.experimental import pallas as pl
from jax.experimental.pallas import tpu as pltpu
from jax.experimental.pallas import tpu_sc as plsc

NC = 2
NS = 16
NW = NC * NS
UNIT = 128
UPC = 4
CHUNK = UNIT * UPC
NBUF = 2


@functools.cache
def _build(R, S, V, D):
    B = R * S
    T4 = D // 8
    n_units = (R // UNIT) * S
    n_chunks_total = n_units // UPC
    n_chunks = n_chunks_total // NW
    mesh = plsc.VectorSubcoreMesh(
        core_axis_name="c", subcore_axis_name="s",
        num_cores=NC, num_subcores=NS,
    )

    def body(instT_hbm, table_hbm, out_hbm, idx_v, rows_v, pox_v,
             isem, gsem, ssem):
        wid = lax.axis_index("s") * NC + lax.axis_index("c")
        chunk0 = wid * n_chunks

        lane = lax.iota(jnp.int32, 16)
        row_ids = [lane + (j * UNIT + l16 * 16)
                   for j in range(UPC) for l16 in range(8)]

        def issue_idx(i, b):
            pltpu.async_copy(
                instT_hbm.at[pl.ds((chunk0 + i) * CHUNK, CHUNK)],
                idx_v.at[b], isem.at[b])

        def wait_idx(b):
            pltpu.make_async_copy(
                instT_hbm.at[pl.ds(0, CHUNK)], idx_v.at[b], isem.at[b]
            ).wait()

        def issue_gather(b):
            pltpu.async_copy(
                table_hbm.at[idx_v.at[b]], rows_v.at[b], gsem.at[b])

        def wait_gather(b):
            pltpu.make_async_copy(
                table_hbm.at[idx_v.at[b]], rows_v.at[b], gsem.at[b]
            ).wait()

        def transpose_chunk(b):
            rows2d = rows_v.at[b]

            @pl.loop(0, D)
            def _(ts):
                col = jnp.full((16,), ts, dtype=jnp.int32)
                tsoff = ts * UNIT
                for j in range(UPC):
                    for l16 in range(8):
                        vals = plsc.load_gather(
                            rows2d, [row_ids[j * 8 + l16], col])
                        pox_v[b, pl.ds(j * (UNIT * D) + tsoff + l16 * 16,
                                       16)] = vals

        def issue_stores(i, b):
            g4 = (chunk0 + i) * UPC
            for j in range(UPC):
                u = g4 + j
                d1 = u >> 7
                b0 = u & (UNIT - 1)
                blk_base = (d1 * T4 * (R // UNIT) + b0) * (8 * UNIT)
                for t in range(T4):
                    pltpu.async_copy(
                        pox_v.at[b, pl.ds((j * T4 + t) * (8 * UNIT),
                                          8 * UNIT)],
                        out_hbm.at[pl.ds(blk_base + t * (R // UNIT) *
                                         (8 * UNIT), 8 * UNIT)],
                        ssem.at[b])

        def wait_stores(b):
            pltpu.make_async_copy(
                out_hbm.at[pl.ds(0, CHUNK * D)], pox_v.at[b], ssem.at[b]
            ).wait()

        issue_idx(0, 0)
        issue_idx(1, 1)
        wait_idx(0)
        issue_gather(0)

        @pl.loop(0, n_chunks, step=NBUF)
        def _(g):
            for b in range(NBUF):
                i = g + b
                nb = 1 - b

                @pl.when(i + 1 < n_chunks)
                def _():
                    wait_idx(nb)
                    issue_gather(nb)

                wait_gather(b)

                @pl.when(i >= NBUF)
                def _():
                    wait_stores(b)

                transpose_chunk(b)
                issue_stores(i, b)

                @pl.when(i + NBUF < n_chunks)
                def _():
                    issue_idx(i + NBUF, b)

        wait_stores(0)
        wait_stores(1)

    return pl.kernel(
        body,
        out_type=jax.ShapeDtypeStruct((B * D,), jnp.float32),
        mesh=mesh,
        compiler_params=pltpu.CompilerParams(use_tc_tiling_on_sc=False,
                                             needs_layout_passes=False),
        scratch_types=[
            pltpu.VMEM((NBUF, CHUNK), jnp.int32),
            pltpu.VMEM((NBUF, CHUNK, D), jnp.float32),
            pltpu.VMEM((NBUF, CHUNK * D), jnp.float32),
            pltpu.SemaphoreType.DMA((NBUF,)),
            pltpu.SemaphoreType.DMA((NBUF,)),
            pltpu.SemaphoreType.DMA((NBUF,)),
        ],
    )


def kernel(inst, table):
    R, S = inst.shape
    V, D = table.shape
    B = R * S
    instT = jnp.swapaxes(inst, 0, 1).reshape(B)
    flat = _build(R, S, V, D)(instT, table)
    out = (flat.reshape(S, D // 8, R // UNIT, 8, UNIT)
               .transpose(2, 4, 0, 1, 3)
               .reshape(R, S, D))
    return out


# speedup vs baseline: 1.2124x; 1.2124x over previous
import functools

import jax
import jax.numpy as jnp
from jax import lax
from jax.experimental import pallas as pl
from jax.experimental.pallas import tpu as pltpu
from jax.experimental.pallas import tpu_sc as plsc

NC = 2
NS = 16
NW = NC * NS
UNIT = 128
UPC = 4
CHUNK = UNIT * UPC
NBUF = 2


@functools.cache
def _build(R, S, V, D):
    B = R * S
    T4 = D // 8
    n_units = (R // UNIT) * S
    n_chunks_total = n_units // UPC
    n_chunks = n_chunks_total // NW
    mesh = plsc.VectorSubcoreMesh(
        core_axis_name="c", subcore_axis_name="s",
        num_cores=NC, num_subcores=NS,
    )

    def body(instT_hbm, table_hbm, out_hbm, idx_v, rows_v, pox_v,
             isem, gsem, ssem):
        wid = lax.axis_index("s") * NC + lax.axis_index("c")
        chunk0 = wid * n_chunks

        lane128 = lax.iota(jnp.int32, 16) * UNIT

        def issue_idx(i, b):
            pltpu.async_copy(
                instT_hbm.at[pl.ds((chunk0 + i) * CHUNK, CHUNK)],
                idx_v.at[b], isem.at[b])

        def wait_idx(b):
            pltpu.make_async_copy(
                instT_hbm.at[pl.ds(0, CHUNK)], idx_v.at[b], isem.at[b]
            ).wait()

        def issue_gather(b):
            pltpu.async_copy(
                table_hbm.at[idx_v.at[b]], rows_v.at[b], gsem.at[b])

        def wait_gather(b):
            pltpu.make_async_copy(
                table_hbm.at[idx_v.at[b]], rows_v.at[b], gsem.at[b]
            ).wait()

        def transpose_chunk(b):
            pox1 = pox_v.at[b]

            @pl.loop(0, CHUNK, unroll=8)
            def _(r):
                j = r >> 7
                l = r & (UNIT - 1)
                d0 = lane128 + jnp.full((16,), (j << 12) | l,
                                        dtype=jnp.int32)
                for h in range(D // 16):
                    vals = rows_v[b, r, pl.ds(h * 16, 16)]
                    plsc.store_scatter(pox1, [d0 + h * (16 * UNIT)], vals)

        def issue_stores(i, b):
            g4 = (chunk0 + i) * UPC
            for j in range(UPC):
                u = g4 + j
                d1 = u >> 7
                b0 = u & (UNIT - 1)
                blk_base = (d1 * T4 * (R // UNIT) + b0) * (8 * UNIT)
                for t in range(T4):
                    pltpu.async_copy(
                        pox_v.at[b, pl.ds((j * T4 + t) * (8 * UNIT),
                                          8 * UNIT)],
                        out_hbm.at[pl.ds(blk_base + t * (R // UNIT) *
                                         (8 * UNIT), 8 * UNIT)],
                        ssem.at[b])

        def wait_stores(b):
            pltpu.make_async_copy(
                out_hbm.at[pl.ds(0, CHUNK * D)], pox_v.at[b], ssem.at[b]
            ).wait()

        issue_idx(0, 0)
        issue_idx(1, 1)
        wait_idx(0)
        issue_gather(0)

        @pl.loop(0, n_chunks, step=NBUF)
        def _(g):
            for b in range(NBUF):
                i = g + b
                nb = 1 - b

                @pl.when(i + 1 < n_chunks)
                def _():
                    wait_idx(nb)
                    issue_gather(nb)

                wait_gather(b)

                @pl.when(i >= NBUF)
                def _():
                    wait_stores(b)

                transpose_chunk(b)
                issue_stores(i, b)

                @pl.when(i + NBUF < n_chunks)
                def _():
                    issue_idx(i + NBUF, b)

        wait_stores(0)
        wait_stores(1)

    return pl.kernel(
        body,
        out_type=jax.ShapeDtypeStruct((B * D,), jnp.float32),
        mesh=mesh,
        compiler_params=pltpu.CompilerParams(use_tc_tiling_on_sc=False,
                                             needs_layout_passes=False),
        scratch_types=[
            pltpu.VMEM((NBUF, CHUNK), jnp.int32),
            pltpu.VMEM((NBUF, CHUNK, D), jnp.float32),
            pltpu.VMEM((NBUF, CHUNK * D), jnp.float32),
            pltpu.SemaphoreType.DMA((NBUF,)),
            pltpu.SemaphoreType.DMA((NBUF,)),
            pltpu.SemaphoreType.DMA((NBUF,)),
        ],
    )


def kernel(inst, table):
    R, S = inst.shape
    V, D = table.shape
    B = R * S
    instT = jnp.swapaxes(inst, 0, 1).reshape(B)
    flat = _build(R, S, V, D)(instT, table)
    out = (flat.reshape(S, D // 8, R // UNIT, 8, UNIT)
               .transpose(2, 4, 0, 1, 3)
               .reshape(R, S, D))
    return out


# speedup vs baseline: 1.6550x; 1.3651x over previous
import functools

import jax
import jax.numpy as jnp
from jax import lax
from jax.experimental import pallas as pl
from jax.experimental.pallas import tpu as pltpu
from jax.experimental.pallas import tpu_sc as plsc

NC = 2
NS = 16
NW = NC * NS
UNIT = 128
UPC = 4
CHUNK = UNIT * UPC
NBUF = 2


@functools.cache
def _build(R, S, V, D):
    B = R * S
    T4 = D // 8
    n_units = (R // UNIT) * S
    n_chunks_total = n_units // UPC
    n_chunks = n_chunks_total // NW
    mesh = plsc.VectorSubcoreMesh(
        core_axis_name="c", subcore_axis_name="s",
        num_cores=NC, num_subcores=NS,
    )

    def body(instT_hbm, table_hbm, out_hbm, idx_v, rows_v, pox_v,
             isem, gsem, ssem):
        wid = lax.axis_index("s") * NC + lax.axis_index("c")
        chunk0 = wid * n_chunks

        lane128 = lax.iota(jnp.int32, 16) * UNIT

        def issue_idx(i, b):
            pltpu.async_copy(
                instT_hbm.at[pl.ds((chunk0 + i) * CHUNK, CHUNK)],
                idx_v.at[b], isem.at[b])

        def wait_idx(b):
            pltpu.make_async_copy(
                instT_hbm.at[pl.ds(0, CHUNK)], idx_v.at[b], isem.at[b]
            ).wait()

        def issue_gather(b):
            pltpu.async_copy(
                table_hbm.at[idx_v.at[b]], rows_v.at[b], gsem.at[b])

        def wait_gather(b):
            pltpu.make_async_copy(
                table_hbm.at[idx_v.at[b]], rows_v.at[b], gsem.at[b]
            ).wait()

        def transpose_chunk(b):
            pox1 = pox_v.at[b]

            @plsc.parallel_loop(0, CHUNK, unroll=8)
            def _(r):
                j = r >> 7
                l = r & (UNIT - 1)
                d0 = lane128 + jnp.full((16,), (j << 12) | l,
                                        dtype=jnp.int32)
                for h in range(D // 16):
                    vals = rows_v[b, r, pl.ds(h * 16, 16)]
                    plsc.store_scatter(pox1, [d0 + h * (16 * UNIT)], vals)

        def issue_stores(i, b):
            g4 = (chunk0 + i) * UPC
            for j in range(UPC):
                u = g4 + j
                d1 = u >> 7
                b0 = u & (UNIT - 1)
                blk_base = (d1 * T4 * (R // UNIT) + b0) * (8 * UNIT)
                for t in range(T4):
                    pltpu.async_copy(
                        pox_v.at[b, pl.ds((j * T4 + t) * (8 * UNIT),
                                          8 * UNIT)],
                        out_hbm.at[pl.ds(blk_base + t * (R // UNIT) *
                                         (8 * UNIT), 8 * UNIT)],
                        ssem.at[b])

        def wait_stores(b):
            pltpu.make_async_copy(
                out_hbm.at[pl.ds(0, CHUNK * D)], pox_v.at[b], ssem.at[b]
            ).wait()

        issue_idx(0, 0)
        issue_idx(1, 1)
        wait_idx(0)
        issue_gather(0)

        @pl.loop(0, n_chunks, step=NBUF)
        def _(g):
            for b in range(NBUF):
                i = g + b
                nb = 1 - b

                @pl.when(i + 1 < n_chunks)
                def _():
                    wait_idx(nb)
                    issue_gather(nb)

                wait_gather(b)

                @pl.when(i >= NBUF)
                def _():
                    wait_stores(b)

                transpose_chunk(b)
                issue_stores(i, b)

                @pl.when(i + NBUF < n_chunks)
                def _():
                    issue_idx(i + NBUF, b)

        wait_stores(0)
        wait_stores(1)

    return pl.kernel(
        body,
        out_type=jax.ShapeDtypeStruct((B * D,), jnp.float32),
        mesh=mesh,
        compiler_params=pltpu.CompilerParams(use_tc_tiling_on_sc=False,
                                             needs_layout_passes=False),
        scratch_types=[
            pltpu.VMEM((NBUF, CHUNK), jnp.int32),
            pltpu.VMEM((NBUF, CHUNK, D), jnp.float32),
            pltpu.VMEM((NBUF, CHUNK * D), jnp.float32),
            pltpu.SemaphoreType.DMA((NBUF,)),
            pltpu.SemaphoreType.DMA((NBUF,)),
            pltpu.SemaphoreType.DMA((NBUF,)),
        ],
    )


def kernel(inst, table):
    R, S = inst.shape
    V, D = table.shape
    B = R * S
    instT = jnp.swapaxes(inst, 0, 1).reshape(B)
    flat = _build(R, S, V, D)(instT, table)
    out = (flat.reshape(S, D // 8, R // UNIT, 8, UNIT)
               .transpose(2, 4, 0, 1, 3)
               .reshape(R, S, D))
    return out
